# baseline (speedup 1.0000x reference)
import functools

import jax
import jax.numpy as jnp
from jax import lax
from jax.experimental import pallas as pl
from jax.experimental.pallas import tpu as pltpu
from jax.experimental.pallas import tpu_sc as plsc

ND = 10000
D = 128
E = 320000
NC = 2
NS = 16
NW = NC * NS
K = 80
CH_TOT = E // K
CHW = CH_TOT // NW
RPT = 624
TAIL = ND - NS * RPT
ZR = 48

_f32 = jnp.float32


def _seg3_body(xa, xb, si_ii, di_ii, si_c, di_c, si_cb, di_cb,
               out_ii, out_c, out_cb,
               acc, zbuf, sidx0, didx0, sidx1, didx1, sidx2, didx2,
               rows0, rows1, rows2,
               smi0, smd0, smi1, smd1, smi2, smd2, smg0, smg1, smg2):
    c = lax.axis_index("c")
    s = lax.axis_index("s")
    wid = c * NS + s
    sbuf = (sidx0, sidx1, sidx2)
    dbuf = (didx0, didx1, didx2)
    rbuf = (rows0, rows1, rows2)
    smi = (smi0, smi1, smi2)
    smd = (smd0, smd1, smd2)
    smg = (smg0, smg1, smg2)

    z16 = jnp.zeros((16,), _f32)

    def zrow(r, carry):
        for j in range(D // 16):
            zbuf[r, pl.ds(j * 16, 16)] = z16
        return carry

    lax.fori_loop(0, ZR, zrow, 0)

    for x_hbm, si_hbm, di_hbm, out_hbm in (
        (xa, si_ii, di_ii, out_ii),
        (xa, si_c, di_c, out_c),
        (xb, si_cb, di_cb, out_cb),
    ):
        cbase = wid * CHW

        def idx_start(ch, b):
            off = pl.multiple_of(jnp.minimum(ch, CH_TOT - 1) * K, 8)
            pltpu.async_copy(si_hbm.at[pl.ds(off, K)], sbuf[b], smi[b])
            pltpu.async_copy(di_hbm.at[pl.ds(off, K)], dbuf[b], smd[b])

        def idx_wait(b):
            pltpu.make_async_copy(si_hbm.at[pl.ds(0, K)], sbuf[b],
                                  smi[b]).wait()
            pltpu.make_async_copy(di_hbm.at[pl.ds(0, K)], dbuf[b],
                                  smd[b]).wait()

        def gather_start(b):
            pltpu.async_copy(x_hbm.at[sbuf[b]], rbuf[b], smg[b])

        def gather_wait(b):
            pltpu.make_async_copy(x_hbm.at[pl.ds(0, K)], rbuf[b],
                                  smg[b]).wait()

        def scat(b):
            pltpu.sync_copy(rbuf[b], acc.at[dbuf[b]], add=True)

        idx_start(cbase, 0)
        idx_start(cbase + 1, 1)
        idx_start(cbase + 2, 2)
        for j in range(RPT // ZR):
            pltpu.sync_copy(zbuf, acc.at[pl.ds(s * RPT + j * ZR, ZR)])

        @pl.when(s == NS - 1)
        def _():
            pltpu.sync_copy(zbuf.at[pl.ds(0, TAIL)],
                            acc.at[pl.ds(NS * RPT, TAIL)])

        idx_wait(0)
        gather_start(0)
        idx_wait(1)
        gather_start(1)
        plsc.subcore_barrier()

        def tri(j, carry):
            i = 3 * j
            for b in range(3):
                gather_wait(b)
                idx_wait((b + 2) % 3)
                gather_start((b + 2) % 3)
                scat(b)
                idx_start(cbase + i + b + 3, b)
            return carry

        lax.fori_loop(0, (CHW - 2) // 3, tri, 0)
        gather_wait(0)
        scat(0)
        gather_wait(1)
        scat(1)
        idx_wait(2)
        plsc.subcore_barrier()

        pltpu.sync_copy(acc.at[pl.ds(s * RPT, RPT)],
                        out_hbm.at[c, pl.ds(s * RPT, RPT)])

        @pl.when(s == NS - 1)
        def _():
            pltpu.sync_copy(acc.at[pl.ds(NS * RPT, TAIL)],
                            out_hbm.at[c, pl.ds(NS * RPT, TAIL)])


def _seg3(xa, xb, si_ii, di_ii, si_c, di_c, si_cb, di_cb):
    mesh = plsc.VectorSubcoreMesh(core_axis_name="c", subcore_axis_name="s")
    f = functools.partial(
        pl.kernel,
        mesh=mesh,
        out_type=[jax.ShapeDtypeStruct((NC, ND, D), _f32)] * 3,
        scratch_types=[
            pltpu.VMEM_SHARED((ND, D), _f32),
            pltpu.VMEM((ZR, D), _f32),
            pltpu.VMEM((K,), jnp.int32),
            pltpu.VMEM((K,), jnp.int32),
            pltpu.VMEM((K,), jnp.int32),
            pltpu.VMEM((K,), jnp.int32),
            pltpu.VMEM((K,), jnp.int32),
            pltpu.VMEM((K,), jnp.int32),
            pltpu.VMEM((K, D), _f32),
            pltpu.VMEM((K, D), _f32),
            pltpu.VMEM((K, D), _f32),
        ] + [pltpu.SemaphoreType.DMA] * 9,
    )(_seg3_body)
    return f(xa, xb, si_ii, di_ii, si_c, di_c, si_cb, di_cb)


_RB = 1000


def _dense_body(aii, acb, ac, xd, xs, wri, wrcb, wrc, wrd, wrs, bd, bs,
                d_o, s_o):
    agg_ii = aii[0] + aii[1]
    agg_cb = acb[0] + acb[1]
    agg_c = ac[0] + ac[1]
    d = (jnp.dot(agg_ii, wri[...], preferred_element_type=_f32)
         + jnp.dot(agg_cb, wrcb[...], preferred_element_type=_f32)
         + jnp.dot(xd[...], wrd[...], preferred_element_type=_f32)
         + bd[...])
    s = (jnp.dot(agg_c, wrc[...], preferred_element_type=_f32)
         + jnp.dot(xs[...], wrs[...], preferred_element_type=_f32)
         + bs[...])
    d_o[...] = jnp.where(d >= 0, d, 0.01 * d)
    s_o[...] = jnp.where(s >= 0, s, 0.01 * s)


def _dense(p_ii, p_cb, p_c, xd, xs, wri, wrcb, wrc, wrd, wrs, bd, bs):
    n = ND // _RB
    part = pl.BlockSpec((2, _RB, D), lambda i: (0, i, 0))
    row = pl.BlockSpec((_RB, D), lambda i: (i, 0))
    mat = pl.BlockSpec((D, D), lambda i: (0, 0))
    vec = pl.BlockSpec((1, D), lambda i: (0, 0))
    return pl.pallas_call(
        _dense_body,
        grid=(n,),
        in_specs=[part, part, part, row, row, mat, mat, mat, mat, mat,
                  vec, vec],
        out_specs=[row, row],
        out_shape=[jax.ShapeDtypeStruct((ND, D), _f32)] * 2,
    )(p_ii, p_cb, p_c, xd, xs, wri, wrcb, wrc, wrd, wrs, bd, bs)


def _bn_body(d2, s2, g, b, d_o, s_o):
    gv = g[...]
    bv = b[...]
    for x, o in ((d2, d_o), (s2, s_o)):
        xv = x[...]
        m = jnp.mean(xv, axis=0, keepdims=True)
        cv = xv - m
        v = jnp.mean(cv * cv, axis=0, keepdims=True)
        o[...] = cv * lax.rsqrt(v + 1e-5) * gv + bv


def _bn(d2, s2, gamma, beta):
    full = pl.BlockSpec((ND, D), lambda: (0, 0))
    vec = pl.BlockSpec((1, D), lambda: (0, 0))
    return pl.pallas_call(
        _bn_body,
        in_specs=[full, full, vec, vec],
        out_specs=[full, full],
        out_shape=[jax.ShapeDtypeStruct((ND, D), _f32)] * 2,
    )(d2, s2, gamma, beta)


def kernel(x_drug, x_se, ei_interacts, ei_causes, ei_caused_by,
           W1_rel_ii, b1_rel_ii, W1_root_ii, W1_rel_c, b1_rel_c, W1_root_c,
           W1_rel_cb, b1_rel_cb, W1_root_cb,
           W2_rel_ii, b2_rel_ii, W2_root_ii, W2_rel_c, b2_rel_c, W2_root_c,
           W2_rel_cb, b2_rel_cb, W2_root_cb, bn_gamma, bn_beta):
    si_ii, di_ii = ei_interacts[0], ei_interacts[1]
    si_c, di_c = ei_causes[0], ei_causes[1]
    si_cb, di_cb = ei_caused_by[0], ei_caused_by[1]

    w1rd = W1_root_ii + W1_root_cb
    b1d = (b1_rel_ii + b1_rel_cb).reshape(1, D)
    b1s = b1_rel_c.reshape(1, D)
    w2rd = W2_root_ii + W2_root_cb
    b2d = (b2_rel_ii + b2_rel_cb).reshape(1, D)
    b2s = b2_rel_c.reshape(1, D)

    p_ii, p_c, p_cb = _seg3(x_drug, x_se, si_ii, di_ii, si_c, di_c,
                            si_cb, di_cb)
    d1, s1 = _dense(p_ii, p_cb, p_c, x_drug, x_se,
                    W1_rel_ii, W1_rel_cb, W1_rel_c, w1rd, W1_root_c,
                    b1d, b1s)
    q_ii, q_c, q_cb = _seg3(d1, s1, si_ii, di_ii, si_c, di_c, si_cb, di_cb)
    d2, s2 = _dense(q_ii, q_cb, q_c, d1, s1,
                    W2_rel_ii, W2_rel_cb, W2_rel_c, w2rd, W2_root_c,
                    b2d, b2s)
    return _bn(d2, s2, bn_gamma.reshape(1, D), bn_beta.reshape(1, D))


# speedup vs baseline: 1.1377x; 1.1377x over previous
import functools

import jax
import jax.numpy as jnp
from jax import lax
from jax.experimental import pallas as pl
from jax.experimental.pallas import tpu as pltpu
from jax.experimental.pallas import tpu_sc as plsc

ND = 10000
D = 128
E = 320000
NC = 2
NS = 16
NW = NC * NS
K = 80
CH_TOT = E // K
CHW = CH_TOT // NW
RPT = 624
TAIL = ND - NS * RPT
ZR = 48

_f32 = jnp.float32


def _seg3_body(xa, xb, si_ii, di_ii, si_c, di_c, si_cb, di_cb,
               out_ii, out_c, out_cb,
               acc, zbuf, sidx0, didx0, sidx1, didx1, sidx2, didx2,
               rows0, rows1, rows2,
               smi0, smd0, smi1, smd1, smi2, smd2, smg0, smg1, smg2):
    c = lax.axis_index("c")
    s = lax.axis_index("s")
    wid = c * NS + s
    sbuf = (sidx0, sidx1, sidx2)
    dbuf = (didx0, didx1, didx2)
    rbuf = (rows0, rows1, rows2)
    smi = (smi0, smi1, smi2)
    smd = (smd0, smd1, smd2)
    smg = (smg0, smg1, smg2)

    z16 = jnp.zeros((16,), _f32)

    def zrow(r, carry):
        for j in range(D // 16):
            zbuf[r, pl.ds(j * 16, 16)] = z16
        return carry

    lax.fori_loop(0, ZR, zrow, 0)

    for x_hbm, si_hbm, di_hbm, out_hbm in (
        (xa, si_ii, di_ii, out_ii),
        (xa, si_c, di_c, out_c),
        (xb, si_cb, di_cb, out_cb),
    ):
        cbase = wid * CHW

        def idx_start(ch, b):
            off = pl.multiple_of(jnp.minimum(ch, CH_TOT - 1) * K, 8)
            pltpu.async_copy(si_hbm.at[pl.ds(off, K)], sbuf[b], smi[b])
            pltpu.async_copy(di_hbm.at[pl.ds(off, K)], dbuf[b], smd[b])

        def idx_wait(b):
            pltpu.make_async_copy(si_hbm.at[pl.ds(0, K)], sbuf[b],
                                  smi[b]).wait()
            pltpu.make_async_copy(di_hbm.at[pl.ds(0, K)], dbuf[b],
                                  smd[b]).wait()

        def gather_start(b):
            pltpu.async_copy(x_hbm.at[sbuf[b]], rbuf[b], smg[b])

        def gather_wait(b):
            pltpu.make_async_copy(x_hbm.at[pl.ds(0, K)], rbuf[b],
                                  smg[b]).wait()

        def scat(b):
            pltpu.sync_copy(rbuf[b], acc.at[dbuf[b]], add=True)

        idx_start(cbase, 0)
        idx_start(cbase + 1, 1)
        idx_start(cbase + 2, 2)
        for j in range(RPT // ZR):
            pltpu.sync_copy(zbuf, acc.at[pl.ds(s * RPT + j * ZR, ZR)])

        @pl.when(s == NS - 1)
        def _():
            pltpu.sync_copy(zbuf.at[pl.ds(0, TAIL)],
                            acc.at[pl.ds(NS * RPT, TAIL)])

        idx_wait(0)
        gather_start(0)
        idx_wait(1)
        gather_start(1)
        plsc.subcore_barrier()

        def tri(j, carry):
            i = 3 * j
            for b in range(3):
                gather_wait(b)
                scat(b)
                idx_wait((b + 2) % 3)
                gather_start((b + 2) % 3)
                idx_start(cbase + i + b + 3, b)
            return carry

        lax.fori_loop(0, (CHW - 2) // 3, tri, 0)
        gather_wait(0)
        scat(0)
        gather_wait(1)
        scat(1)
        idx_wait(2)
        plsc.subcore_barrier()

        pltpu.sync_copy(acc.at[pl.ds(s * RPT, RPT)],
                        out_hbm.at[c, pl.ds(s * RPT, RPT)])

        @pl.when(s == NS - 1)
        def _():
            pltpu.sync_copy(acc.at[pl.ds(NS * RPT, TAIL)],
                            out_hbm.at[c, pl.ds(NS * RPT, TAIL)])


def _seg3(xa, xb, si_ii, di_ii, si_c, di_c, si_cb, di_cb):
    mesh = plsc.VectorSubcoreMesh(core_axis_name="c", subcore_axis_name="s")
    f = functools.partial(
        pl.kernel,
        mesh=mesh,
        out_type=[jax.ShapeDtypeStruct((NC, ND, D), _f32)] * 3,
        scratch_types=[
            pltpu.VMEM_SHARED((ND, D), _f32),
            pltpu.VMEM((ZR, D), _f32),
            pltpu.VMEM((K,), jnp.int32),
            pltpu.VMEM((K,), jnp.int32),
            pltpu.VMEM((K,), jnp.int32),
            pltpu.VMEM((K,), jnp.int32),
            pltpu.VMEM((K,), jnp.int32),
            pltpu.VMEM((K,), jnp.int32),
            pltpu.VMEM((K, D), _f32),
            pltpu.VMEM((K, D), _f32),
            pltpu.VMEM((K, D), _f32),
        ] + [pltpu.SemaphoreType.DMA] * 9,
    )(_seg3_body)
    return f(xa, xb, si_ii, di_ii, si_c, di_c, si_cb, di_cb)


_RB = 1000


def _dense_body(aii, acb, ac, xd, xs, wri, wrcb, wrc, wrd, wrs, bd, bs,
                d_o, s_o):
    agg_ii = aii[0] + aii[1]
    agg_cb = acb[0] + acb[1]
    agg_c = ac[0] + ac[1]
    d = (jnp.dot(agg_ii, wri[...], preferred_element_type=_f32)
         + jnp.dot(agg_cb, wrcb[...], preferred_element_type=_f32)
         + jnp.dot(xd[...], wrd[...], preferred_element_type=_f32)
         + bd[...])
    s = (jnp.dot(agg_c, wrc[...], preferred_element_type=_f32)
         + jnp.dot(xs[...], wrs[...], preferred_element_type=_f32)
         + bs[...])
    d_o[...] = jnp.where(d >= 0, d, 0.01 * d)
    s_o[...] = jnp.where(s >= 0, s, 0.01 * s)


def _dense(p_ii, p_cb, p_c, xd, xs, wri, wrcb, wrc, wrd, wrs, bd, bs):
    n = ND // _RB
    part = pl.BlockSpec((2, _RB, D), lambda i: (0, i, 0))
    row = pl.BlockSpec((_RB, D), lambda i: (i, 0))
    mat = pl.BlockSpec((D, D), lambda i: (0, 0))
    vec = pl.BlockSpec((1, D), lambda i: (0, 0))
    return pl.pallas_call(
        _dense_body,
        grid=(n,),
        in_specs=[part, part, part, row, row, mat, mat, mat, mat, mat,
                  vec, vec],
        out_specs=[row, row],
        out_shape=[jax.ShapeDtypeStruct((ND, D), _f32)] * 2,
    )(p_ii, p_cb, p_c, xd, xs, wri, wrcb, wrc, wrd, wrs, bd, bs)


def _bn_body(d2, s2, g, b, d_o, s_o):
    gv = g[...]
    bv = b[...]
    for x, o in ((d2, d_o), (s2, s_o)):
        xv = x[...]
        m = jnp.mean(xv, axis=0, keepdims=True)
        cv = xv - m
        v = jnp.mean(cv * cv, axis=0, keepdims=True)
        o[...] = cv * lax.rsqrt(v + 1e-5) * gv + bv


def _bn(d2, s2, gamma, beta):
    full = pl.BlockSpec((ND, D), lambda: (0, 0))
    vec = pl.BlockSpec((1, D), lambda: (0, 0))
    return pl.pallas_call(
        _bn_body,
        in_specs=[full, full, vec, vec],
        out_specs=[full, full],
        out_shape=[jax.ShapeDtypeStruct((ND, D), _f32)] * 2,
    )(d2, s2, gamma, beta)


def kernel(x_drug, x_se, ei_interacts, ei_causes, ei_caused_by,
           W1_rel_ii, b1_rel_ii, W1_root_ii, W1_rel_c, b1_rel_c, W1_root_c,
           W1_rel_cb, b1_rel_cb, W1_root_cb,
           W2_rel_ii, b2_rel_ii, W2_root_ii, W2_rel_c, b2_rel_c, W2_root_c,
           W2_rel_cb, b2_rel_cb, W2_root_cb, bn_gamma, bn_beta):
    si_ii, di_ii = ei_interacts[0], ei_interacts[1]
    si_c, di_c = ei_causes[0], ei_causes[1]
    si_cb, di_cb = ei_caused_by[0], ei_caused_by[1]

    w1rd = W1_root_ii + W1_root_cb
    b1d = (b1_rel_ii + b1_rel_cb).reshape(1, D)
    b1s = b1_rel_c.reshape(1, D)
    w2rd = W2_root_ii + W2_root_cb
    b2d = (b2_rel_ii + b2_rel_cb).reshape(1, D)
    b2s = b2_rel_c.reshape(1, D)

    p_ii, p_c, p_cb = _seg3(x_drug, x_se, si_ii, di_ii, si_c, di_c,
                            si_cb, di_cb)
    d1, s1 = _dense(p_ii, p_cb, p_c, x_drug, x_se,
                    W1_rel_ii, W1_rel_cb, W1_rel_c, w1rd, W1_root_c,
                    b1d, b1s)
    q_ii, q_c, q_cb = _seg3(d1, s1, si_ii, di_ii, si_c, di_c, si_cb, di_cb)
    d2, s2 = _dense(q_ii, q_cb, q_c, d1, s1,
                    W2_rel_ii, W2_rel_cb, W2_rel_c, w2rd, W2_root_c,
                    b2d, b2s)
    return _bn(d2, s2, bn_gamma.reshape(1, D), bn_beta.reshape(1, D))


# speedup vs baseline: 1.3083x; 1.1499x over previous
import functools

import jax
import jax.numpy as jnp
from jax import lax
from jax.experimental import pallas as pl
from jax.experimental.pallas import tpu as pltpu
from jax.experimental.pallas import tpu_sc as plsc

ND = 10000
D = 128
E = 320000
NC = 2
NS = 16
NW = NC * NS
K = 80
CH_TOT = E // K
CHW = CH_TOT // NW
RPT = 624
TAIL = ND - NS * RPT
ZR = 48

_f32 = jnp.float32


def _seg3_body(xa, xb, si_ii, di_ii, si_c, di_c, si_cb, di_cb,
               out_ii, out_c, out_cb,
               acc, zbuf, sidx0, didx0, sidx1, didx1, sidx2, didx2,
               sidx3, didx3, rows0, rows1, rows2, rows3,
               smi0, smd0, smi1, smd1, smi2, smd2, smi3, smd3,
               smg0, smg1, smg2, smg3, sms0, sms1, sms2, sms3):
    c = lax.axis_index("c")
    s = lax.axis_index("s")
    wid = c * NS + s
    sbuf = (sidx0, sidx1, sidx2, sidx3)
    dbuf = (didx0, didx1, didx2, didx3)
    rbuf = (rows0, rows1, rows2, rows3)
    smi = (smi0, smi1, smi2, smi3)
    smd = (smd0, smd1, smd2, smd3)
    smg = (smg0, smg1, smg2, smg3)
    sms = (sms0, sms1, sms2, sms3)

    z16 = jnp.zeros((16,), _f32)

    def zrow(r, carry):
        for j in range(D // 16):
            zbuf[r, pl.ds(j * 16, 16)] = z16
        return carry

    lax.fori_loop(0, ZR, zrow, 0)

    for x_hbm, si_hbm, di_hbm, out_hbm in (
        (xa, si_ii, di_ii, out_ii),
        (xa, si_c, di_c, out_c),
        (xb, si_cb, di_cb, out_cb),
    ):
        cbase = wid * CHW

        def idx_start(ch, b):
            off = pl.multiple_of(jnp.minimum(ch, CH_TOT - 1) * K, 8)
            pltpu.async_copy(si_hbm.at[pl.ds(off, K)], sbuf[b], smi[b])
            pltpu.async_copy(di_hbm.at[pl.ds(off, K)], dbuf[b], smd[b])

        def idx_wait(b):
            pltpu.make_async_copy(si_hbm.at[pl.ds(0, K)], sbuf[b],
                                  smi[b]).wait()
            pltpu.make_async_copy(di_hbm.at[pl.ds(0, K)], dbuf[b],
                                  smd[b]).wait()

        def gather_start(b):
            pltpu.async_copy(x_hbm.at[sbuf[b]], rbuf[b], smg[b])

        def gather_wait(b):
            pltpu.make_async_copy(x_hbm.at[pl.ds(0, K)], rbuf[b],
                                  smg[b]).wait()

        def scat_start(b):
            pltpu.async_copy(rbuf[b], acc.at[dbuf[b]], sms[b], add=True)

        def scat_wait(b):
            pltpu.make_async_copy(rbuf[b], acc.at[pl.ds(0, K)],
                                  sms[b]).wait()

        idx_start(cbase, 0)
        idx_start(cbase + 1, 1)
        idx_start(cbase + 2, 2)
        for j in range(RPT // ZR):
            pltpu.sync_copy(zbuf, acc.at[pl.ds(s * RPT + j * ZR, ZR)])

        @pl.when(s == NS - 1)
        def _():
            pltpu.sync_copy(zbuf.at[pl.ds(0, TAIL)],
                            acc.at[pl.ds(NS * RPT, TAIL)])

        idx_wait(0)
        gather_start(0)
        idx_wait(1)
        gather_start(1)
        plsc.subcore_barrier()

        gather_wait(0)
        scat_start(0)
        idx_start(cbase + 3, 3)
        idx_wait(2)
        gather_start(2)

        def quad(j, carry):
            i = 4 * j + 1
            for bp in range(4):
                ch = i + bp
                b = (1 + bp) % 4
                gather_wait(b)
                scat_wait((b + 3) % 4)
                scat_start(b)
                idx_start(cbase + ch + 3, (b + 3) % 4)
                idx_wait((b + 2) % 4)
                gather_start((b + 2) % 4)
            return carry

        lax.fori_loop(0, (CHW - 1) // 4, quad, 0)
        scat_wait((CHW - 1) % 4)
        gather_wait(CHW % 4)
        gather_wait((CHW + 1) % 4)
        idx_wait((CHW + 2) % 4)
        plsc.subcore_barrier()

        pltpu.sync_copy(acc.at[pl.ds(s * RPT, RPT)],
                        out_hbm.at[c, pl.ds(s * RPT, RPT)])

        @pl.when(s == NS - 1)
        def _():
            pltpu.sync_copy(acc.at[pl.ds(NS * RPT, TAIL)],
                            out_hbm.at[c, pl.ds(NS * RPT, TAIL)])


def _seg3(xa, xb, si_ii, di_ii, si_c, di_c, si_cb, di_cb):
    mesh = plsc.VectorSubcoreMesh(core_axis_name="c", subcore_axis_name="s")
    f = functools.partial(
        pl.kernel,
        mesh=mesh,
        out_type=[jax.ShapeDtypeStruct((NC, ND, D), _f32)] * 3,
        scratch_types=[
            pltpu.VMEM_SHARED((ND, D), _f32),
            pltpu.VMEM((ZR, D), _f32),
            pltpu.VMEM((K,), jnp.int32),
            pltpu.VMEM((K,), jnp.int32),
            pltpu.VMEM((K,), jnp.int32),
            pltpu.VMEM((K,), jnp.int32),
            pltpu.VMEM((K,), jnp.int32),
            pltpu.VMEM((K,), jnp.int32),
            pltpu.VMEM((K,), jnp.int32),
            pltpu.VMEM((K,), jnp.int32),
            pltpu.VMEM((K, D), _f32),
            pltpu.VMEM((K, D), _f32),
            pltpu.VMEM((K, D), _f32),
            pltpu.VMEM((K, D), _f32),
        ] + [pltpu.SemaphoreType.DMA] * 16,
    )(_seg3_body)
    return f(xa, xb, si_ii, di_ii, si_c, di_c, si_cb, di_cb)


_RB = 1000


def _dense_body(aii, acb, ac, xd, xs, wri, wrcb, wrc, wrd, wrs, bd, bs,
                d_o, s_o):
    agg_ii = aii[0] + aii[1]
    agg_cb = acb[0] + acb[1]
    agg_c = ac[0] + ac[1]
    d = (jnp.dot(agg_ii, wri[...], preferred_element_type=_f32)
         + jnp.dot(agg_cb, wrcb[...], preferred_element_type=_f32)
         + jnp.dot(xd[...], wrd[...], preferred_element_type=_f32)
         + bd[...])
    s = (jnp.dot(agg_c, wrc[...], preferred_element_type=_f32)
         + jnp.dot(xs[...], wrs[...], preferred_element_type=_f32)
         + bs[...])
    d_o[...] = jnp.where(d >= 0, d, 0.01 * d)
    s_o[...] = jnp.where(s >= 0, s, 0.01 * s)


def _dense(p_ii, p_cb, p_c, xd, xs, wri, wrcb, wrc, wrd, wrs, bd, bs):
    n = ND // _RB
    part = pl.BlockSpec((2, _RB, D), lambda i: (0, i, 0))
    row = pl.BlockSpec((_RB, D), lambda i: (i, 0))
    mat = pl.BlockSpec((D, D), lambda i: (0, 0))
    vec = pl.BlockSpec((1, D), lambda i: (0, 0))
    return pl.pallas_call(
        _dense_body,
        grid=(n,),
        in_specs=[part, part, part, row, row, mat, mat, mat, mat, mat,
                  vec, vec],
        out_specs=[row, row],
        out_shape=[jax.ShapeDtypeStruct((ND, D), _f32)] * 2,
    )(p_ii, p_cb, p_c, xd, xs, wri, wrcb, wrc, wrd, wrs, bd, bs)


def _bn_body(d2, s2, g, b, d_o, s_o):
    gv = g[...]
    bv = b[...]
    for x, o in ((d2, d_o), (s2, s_o)):
        xv = x[...]
        m = jnp.mean(xv, axis=0, keepdims=True)
        cv = xv - m
        v = jnp.mean(cv * cv, axis=0, keepdims=True)
        o[...] = cv * lax.rsqrt(v + 1e-5) * gv + bv


def _bn(d2, s2, gamma, beta):
    full = pl.BlockSpec((ND, D), lambda: (0, 0))
    vec = pl.BlockSpec((1, D), lambda: (0, 0))
    return pl.pallas_call(
        _bn_body,
        in_specs=[full, full, vec, vec],
        out_specs=[full, full],
        out_shape=[jax.ShapeDtypeStruct((ND, D), _f32)] * 2,
    )(d2, s2, gamma, beta)


def kernel(x_drug, x_se, ei_interacts, ei_causes, ei_caused_by,
           W1_rel_ii, b1_rel_ii, W1_root_ii, W1_rel_c, b1_rel_c, W1_root_c,
           W1_rel_cb, b1_rel_cb, W1_root_cb,
           W2_rel_ii, b2_rel_ii, W2_root_ii, W2_rel_c, b2_rel_c, W2_root_c,
           W2_rel_cb, b2_rel_cb, W2_root_cb, bn_gamma, bn_beta):
    si_ii, di_ii = ei_interacts[0], ei_interacts[1]
    si_c, di_c = ei_causes[0], ei_causes[1]
    si_cb, di_cb = ei_caused_by[0], ei_caused_by[1]

    w1rd = W1_root_ii + W1_root_cb
    b1d = (b1_rel_ii + b1_rel_cb).reshape(1, D)
    b1s = b1_rel_c.reshape(1, D)
    w2rd = W2_root_ii + W2_root_cb
    b2d = (b2_rel_ii + b2_rel_cb).reshape(1, D)
    b2s = b2_rel_c.reshape(1, D)

    p_ii, p_c, p_cb = _seg3(x_drug, x_se, si_ii, di_ii, si_c, di_c,
                            si_cb, di_cb)
    d1, s1 = _dense(p_ii, p_cb, p_c, x_drug, x_se,
                    W1_rel_ii, W1_rel_cb, W1_rel_c, w1rd, W1_root_c,
                    b1d, b1s)
    q_ii, q_c, q_cb = _seg3(d1, s1, si_ii, di_ii, si_c, di_c, si_cb, di_cb)
    d2, s2 = _dense(q_ii, q_cb, q_c, d1, s1,
                    W2_rel_ii, W2_rel_cb, W2_rel_c, w2rd, W2_root_c,
                    b2d, b2s)
    return _bn(d2, s2, bn_gamma.reshape(1, D), bn_beta.reshape(1, D))


# speedup vs baseline: 1.3225x; 1.0108x over previous
import functools

import jax
import jax.numpy as jnp
from jax import lax
from jax.experimental import pallas as pl
from jax.experimental.pallas import tpu as pltpu
from jax.experimental.pallas import tpu_sc as plsc

ND = 10000
D = 128
E = 320000
NC = 2
NS = 16
NW = NC * NS
K = 80
CH_TOT = E // K
CHW = CH_TOT // NW
RPT = 624
TAIL = ND - NS * RPT
ZR = 48

_f32 = jnp.float32


def _seg3_body(xa, xb, si_ii, di_ii, si_c, di_c, si_cb, di_cb,
               out_ii, out_c, out_cb,
               acc, zbuf, sidx0, didx0, sidx1, didx1, sidx2, didx2,
               sidx3, didx3, rows0, rows1, rows2, rows3,
               smi0, smd0, smi1, smd1, smi2, smd2, smi3, smd3,
               smg0, smg1, smg2, smg3, sms0, sms1, sms2, sms3):
    c = lax.axis_index("c")
    s = lax.axis_index("s")
    wid = c * NS + s
    sbuf = (sidx0, sidx1, sidx2, sidx3)
    dbuf = (didx0, didx1, didx2, didx3)
    rbuf = (rows0, rows1, rows2, rows3)
    smi = (smi0, smi1, smi2, smi3)
    smd = (smd0, smd1, smd2, smd3)
    smg = (smg0, smg1, smg2, smg3)
    sms = (sms0, sms1, sms2, sms3)

    z16 = jnp.zeros((16,), _f32)

    def zrow(r, carry):
        for j in range(D // 16):
            zbuf[r, pl.ds(j * 16, 16)] = z16
        return carry

    lax.fori_loop(0, ZR, zrow, 0)

    for x_hbm, si_hbm, di_hbm, out_hbm in (
        (xa, si_ii, di_ii, out_ii),
        (xa, si_c, di_c, out_c),
        (xb, si_cb, di_cb, out_cb),
    ):
        cbase = wid * CHW

        def idx_start(ch, b):
            off = pl.multiple_of(jnp.minimum(ch, CH_TOT - 1) * K, 8)
            pltpu.async_copy(si_hbm.at[pl.ds(off, K)], sbuf[b], smi[b])
            pltpu.async_copy(di_hbm.at[pl.ds(off, K)], dbuf[b], smd[b])

        def idx_wait(b):
            pltpu.make_async_copy(si_hbm.at[pl.ds(0, K)], sbuf[b],
                                  smi[b]).wait()
            pltpu.make_async_copy(di_hbm.at[pl.ds(0, K)], dbuf[b],
                                  smd[b]).wait()

        def gather_start(b):
            pltpu.async_copy(x_hbm.at[sbuf[b]], rbuf[b], smg[b])

        def gather_wait(b):
            pltpu.make_async_copy(x_hbm.at[pl.ds(0, K)], rbuf[b],
                                  smg[b]).wait()

        def scat_start(b):
            pltpu.async_copy(rbuf[b], acc.at[dbuf[b]], sms[b], add=True)

        def scat_wait(b):
            pltpu.make_async_copy(rbuf[b], acc.at[pl.ds(0, K)],
                                  sms[b]).wait()

        idx_start(cbase, 0)
        idx_start(cbase + 1, 1)
        idx_start(cbase + 2, 2)
        for j in range(RPT // ZR):
            pltpu.sync_copy(zbuf, acc.at[pl.ds(s * RPT + j * ZR, ZR)])

        @pl.when(s == NS - 1)
        def _():
            pltpu.sync_copy(zbuf.at[pl.ds(0, TAIL)],
                            acc.at[pl.ds(NS * RPT, TAIL)])

        idx_wait(0)
        gather_start(0)
        idx_wait(1)
        gather_start(1)
        plsc.subcore_barrier()

        gather_wait(0)
        scat_start(0)
        idx_start(cbase + 3, 3)
        idx_wait(2)
        gather_start(2)

        def quad(j, carry):
            i = 4 * j + 1
            for bp in range(4):
                ch = i + bp
                b = (1 + bp) % 4
                gather_wait(b)
                scat_wait((b + 3) % 4)
                scat_start(b)
                idx_start(cbase + ch + 3, (b + 3) % 4)
                idx_wait((b + 2) % 4)
                gather_start((b + 2) % 4)
            return carry

        lax.fori_loop(0, (CHW - 1) // 4, quad, 0)
        scat_wait((CHW - 1) % 4)
        gather_wait(CHW % 4)
        gather_wait((CHW + 1) % 4)
        idx_wait((CHW + 2) % 4)
        plsc.subcore_barrier()

        pltpu.sync_copy(acc.at[pl.ds(s * RPT, RPT)],
                        out_hbm.at[c, pl.ds(s * RPT, RPT)])

        @pl.when(s == NS - 1)
        def _():
            pltpu.sync_copy(acc.at[pl.ds(NS * RPT, TAIL)],
                            out_hbm.at[c, pl.ds(NS * RPT, TAIL)])


def _seg3(xa, xb, si_ii, di_ii, si_c, di_c, si_cb, di_cb):
    mesh = plsc.VectorSubcoreMesh(core_axis_name="c", subcore_axis_name="s")
    f = functools.partial(
        pl.kernel,
        mesh=mesh,
        out_type=[jax.ShapeDtypeStruct((NC, ND, D), _f32)] * 3,
        scratch_types=[
            pltpu.VMEM_SHARED((ND, D), _f32),
            pltpu.VMEM((ZR, D), _f32),
            pltpu.VMEM((K,), jnp.int32),
            pltpu.VMEM((K,), jnp.int32),
            pltpu.VMEM((K,), jnp.int32),
            pltpu.VMEM((K,), jnp.int32),
            pltpu.VMEM((K,), jnp.int32),
            pltpu.VMEM((K,), jnp.int32),
            pltpu.VMEM((K,), jnp.int32),
            pltpu.VMEM((K,), jnp.int32),
            pltpu.VMEM((K, D), _f32),
            pltpu.VMEM((K, D), _f32),
            pltpu.VMEM((K, D), _f32),
            pltpu.VMEM((K, D), _f32),
        ] + [pltpu.SemaphoreType.DMA] * 16,
    )(_seg3_body)
    return f(xa, xb, si_ii, di_ii, si_c, di_c, si_cb, di_cb)


_RB = 1000


def _dense_body(aii, acb, ac, xd, xs, wri, wrcb, wrc, wrd, wrs, bd, bs,
                d_o, s_o):
    agg_ii = aii[0] + aii[1]
    agg_cb = acb[0] + acb[1]
    agg_c = ac[0] + ac[1]
    d = (jnp.dot(agg_ii, wri[...], preferred_element_type=_f32)
         + jnp.dot(agg_cb, wrcb[...], preferred_element_type=_f32)
         + jnp.dot(xd[...], wrd[...], preferred_element_type=_f32)
         + bd[...])
    s = (jnp.dot(agg_c, wrc[...], preferred_element_type=_f32)
         + jnp.dot(xs[...], wrs[...], preferred_element_type=_f32)
         + bs[...])
    d_o[...] = jnp.where(d >= 0, d, 0.01 * d)
    s_o[...] = jnp.where(s >= 0, s, 0.01 * s)


def _dense(p_ii, p_cb, p_c, xd, xs, wri, wrcb, wrc, wrd, wrs, bd, bs):
    n = ND // _RB
    part = pl.BlockSpec((2, _RB, D), lambda i: (0, i, 0))
    row = pl.BlockSpec((_RB, D), lambda i: (i, 0))
    mat = pl.BlockSpec((D, D), lambda i: (0, 0))
    vec = pl.BlockSpec((1, D), lambda i: (0, 0))
    return pl.pallas_call(
        _dense_body,
        grid=(n,),
        in_specs=[part, part, part, row, row, mat, mat, mat, mat, mat,
                  vec, vec],
        out_specs=[row, row],
        out_shape=[jax.ShapeDtypeStruct((ND, D), _f32)] * 2,
    )(p_ii, p_cb, p_c, xd, xs, wri, wrcb, wrc, wrd, wrs, bd, bs)


_NB = ND // _RB


def _dense2bn_body(aii, acb, ac, xd, xs, wri, wrcb, wrc, wrd, wrs, bd, bs,
                   g, bb, d_o, s_o, draw, sraw, stat):
    i = pl.program_id(0)

    @pl.when(i < _NB)
    def _():
        agg_ii = aii[0] + aii[1]
        agg_cb = acb[0] + acb[1]
        agg_c = ac[0] + ac[1]
        d = (jnp.dot(agg_ii, wri[...], preferred_element_type=_f32)
             + jnp.dot(agg_cb, wrcb[...], preferred_element_type=_f32)
             + jnp.dot(xd[...], wrd[...], preferred_element_type=_f32)
             + bd[...])
        s = (jnp.dot(agg_c, wrc[...], preferred_element_type=_f32)
             + jnp.dot(xs[...], wrs[...], preferred_element_type=_f32)
             + bs[...])
        d = jnp.where(d >= 0, d, 0.01 * d)
        s = jnp.where(s >= 0, s, 0.01 * s)

        @pl.when(i == 0)
        def _():
            stat[...] = jnp.zeros((8, D), _f32)

        draw[pl.ds(i * _RB, _RB), :] = d
        sraw[pl.ds(i * _RB, _RB), :] = s
        stat[0:1, :] += jnp.sum(d, axis=0, keepdims=True)
        stat[1:2, :] += jnp.sum(d * d, axis=0, keepdims=True)
        stat[2:3, :] += jnp.sum(s, axis=0, keepdims=True)
        stat[3:4, :] += jnp.sum(s * s, axis=0, keepdims=True)

    @pl.when(i >= _NB)
    def _():
        k = i - _NB
        gv = g[...]
        bv = bb[...]
        for raw, r0, o in ((draw, 0, d_o), (sraw, 2, s_o)):
            xv = raw[pl.ds(k * _RB, _RB), :]
            m = stat[r0:r0 + 1, :] * (1.0 / ND)
            v = stat[r0 + 1:r0 + 2, :] * (1.0 / ND) - m * m
            o[...] = (xv - m) * lax.rsqrt(v + 1e-5) * gv + bv


def _dense2bn(p_ii, p_cb, p_c, xd, xs, wri, wrcb, wrc, wrd, wrs, bd, bs,
              gamma, beta):
    part = pl.BlockSpec((2, _RB, D), lambda i: (0, jnp.minimum(i, _NB - 1), 0))
    row = pl.BlockSpec((_RB, D), lambda i: (jnp.minimum(i, _NB - 1), 0))
    mat = pl.BlockSpec((D, D), lambda i: (0, 0))
    vec = pl.BlockSpec((1, D), lambda i: (0, 0))
    orow = pl.BlockSpec((_RB, D), lambda i: (jnp.maximum(i - _NB, 0), 0))
    return pl.pallas_call(
        _dense2bn_body,
        grid=(2 * _NB,),
        in_specs=[part, part, part, row, row, mat, mat, mat, mat, mat,
                  vec, vec, vec, vec],
        out_specs=[orow, orow],
        out_shape=[jax.ShapeDtypeStruct((ND, D), _f32)] * 2,
        scratch_shapes=[
            pltpu.VMEM((ND, D), _f32),
            pltpu.VMEM((ND, D), _f32),
            pltpu.VMEM((8, D), _f32),
        ],
    )(p_ii, p_cb, p_c, xd, xs, wri, wrcb, wrc, wrd, wrs, bd, bs,
      gamma, beta)


def kernel(x_drug, x_se, ei_interacts, ei_causes, ei_caused_by,
           W1_rel_ii, b1_rel_ii, W1_root_ii, W1_rel_c, b1_rel_c, W1_root_c,
           W1_rel_cb, b1_rel_cb, W1_root_cb,
           W2_rel_ii, b2_rel_ii, W2_root_ii, W2_rel_c, b2_rel_c, W2_root_c,
           W2_rel_cb, b2_rel_cb, W2_root_cb, bn_gamma, bn_beta):
    si_ii, di_ii = ei_interacts[0], ei_interacts[1]
    si_c, di_c = ei_causes[0], ei_causes[1]
    si_cb, di_cb = ei_caused_by[0], ei_caused_by[1]

    w1rd = W1_root_ii + W1_root_cb
    b1d = (b1_rel_ii + b1_rel_cb).reshape(1, D)
    b1s = b1_rel_c.reshape(1, D)
    w2rd = W2_root_ii + W2_root_cb
    b2d = (b2_rel_ii + b2_rel_cb).reshape(1, D)
    b2s = b2_rel_c.reshape(1, D)

    p_ii, p_c, p_cb = _seg3(x_drug, x_se, si_ii, di_ii, si_c, di_c,
                            si_cb, di_cb)
    d1, s1 = _dense(p_ii, p_cb, p_c, x_drug, x_se,
                    W1_rel_ii, W1_rel_cb, W1_rel_c, w1rd, W1_root_c,
                    b1d, b1s)
    q_ii, q_c, q_cb = _seg3(d1, s1, si_ii, di_ii, si_c, di_c, si_cb, di_cb)
    return _dense2bn(q_ii, q_cb, q_c, d1, s1,
                     W2_rel_ii, W2_rel_cb, W2_rel_c, w2rd, W2_root_c,
                     b2d, b2s,
                     bn_gamma.reshape(1, D), bn_beta.reshape(1, D))


# speedup vs baseline: 1.3309x; 1.0064x over previous
import functools

import jax
import jax.numpy as jnp
from jax import lax
from jax.experimental import pallas as pl
from jax.experimental.pallas import tpu as pltpu
from jax.experimental.pallas import tpu_sc as plsc

ND = 10000
D = 128
E = 320000
NC = 2
NS = 16
NW = NC * NS
K = 80
CH_TOT = E // K
CHW = CH_TOT // NW
RPT = 624
TAIL = ND - NS * RPT
ZR = 48

_f32 = jnp.float32


def _seg3_body(xa, xb, si_ii, di_ii, si_c, di_c, si_cb, di_cb,
               out_ii, out_c, out_cb,
               acc, zbuf, sidx0, didx0, sidx1, didx1, sidx2, didx2,
               sidx3, didx3, rows0, rows1, rows2, rows3,
               smi0, smd0, smi1, smd1, smi2, smd2, smi3, smd3,
               smg0, smg1, smg2, smg3, sms0, sms1, sms2, sms3):
    c = lax.axis_index("c")
    s = lax.axis_index("s")
    wid = c * NS + s
    sbuf = (sidx0, sidx1, sidx2, sidx3)
    dbuf = (didx0, didx1, didx2, didx3)
    rbuf = (rows0, rows1, rows2, rows3)
    smi = (smi0, smi1, smi2, smi3)
    smd = (smd0, smd1, smd2, smd3)
    smg = (smg0, smg1, smg2, smg3)
    sms = (sms0, sms1, sms2, sms3)

    z16 = jnp.zeros((16,), _f32)

    def zrow(r, carry):
        for j in range(D // 16):
            zbuf[r, pl.ds(j * 16, 16)] = z16
        return carry

    lax.fori_loop(0, ZR, zrow, 0)

    prev_out = []
    for x_hbm, si_hbm, di_hbm, out_hbm in (
        (xa, si_ii, di_ii, out_ii),
        (xa, si_c, di_c, out_c),
        (xb, si_cb, di_cb, out_cb),
    ):
        cbase = wid * CHW

        def idx_start(ch, b):
            off = pl.multiple_of(jnp.minimum(ch, CH_TOT - 1) * K, 8)
            pltpu.async_copy(si_hbm.at[pl.ds(off, K)], sbuf[b], smi[b])
            pltpu.async_copy(di_hbm.at[pl.ds(off, K)], dbuf[b], smd[b])

        def idx_wait(b):
            pltpu.make_async_copy(si_hbm.at[pl.ds(0, K)], sbuf[b],
                                  smi[b]).wait()
            pltpu.make_async_copy(di_hbm.at[pl.ds(0, K)], dbuf[b],
                                  smd[b]).wait()

        def gather_start(b):
            pltpu.async_copy(x_hbm.at[sbuf[b]], rbuf[b], smg[b])

        def gather_wait(b):
            pltpu.make_async_copy(x_hbm.at[pl.ds(0, K)], rbuf[b],
                                  smg[b]).wait()

        def scat_start(b):
            pltpu.async_copy(rbuf[b], acc.at[dbuf[b]], sms[b], add=True)

        def scat_wait(b):
            pltpu.make_async_copy(rbuf[b], acc.at[pl.ds(0, K)],
                                  sms[b]).wait()

        def dump_wait(o):
            pltpu.make_async_copy(acc.at[pl.ds(s * RPT, RPT)],
                                  o.at[c, pl.ds(s * RPT, RPT)],
                                  smg[0]).wait()

            @pl.when(s == NS - 1)
            def _():
                pltpu.make_async_copy(acc.at[pl.ds(NS * RPT, TAIL)],
                                      o.at[c, pl.ds(NS * RPT, TAIL)],
                                      smg[1]).wait()

        idx_start(cbase, 0)
        idx_start(cbase + 1, 1)
        idx_start(cbase + 2, 2)
        if prev_out:
            dump_wait(prev_out.pop())
        for j in range(RPT // ZR):
            pltpu.sync_copy(zbuf, acc.at[pl.ds(s * RPT + j * ZR, ZR)])

        @pl.when(s == NS - 1)
        def _():
            pltpu.sync_copy(zbuf.at[pl.ds(0, TAIL)],
                            acc.at[pl.ds(NS * RPT, TAIL)])

        idx_wait(0)
        gather_start(0)
        idx_wait(1)
        gather_start(1)
        plsc.subcore_barrier()

        gather_wait(0)
        scat_start(0)
        idx_start(cbase + 3, 3)
        idx_wait(2)
        gather_start(2)

        def quad(j, carry):
            i = 4 * j + 1
            for bp in range(4):
                ch = i + bp
                b = (1 + bp) % 4
                gather_wait(b)
                scat_wait((b + 3) % 4)
                scat_start(b)
                idx_start(cbase + ch + 3, (b + 3) % 4)
                idx_wait((b + 2) % 4)
                gather_start((b + 2) % 4)
            return carry

        lax.fori_loop(0, (CHW - 1) // 4, quad, 0)
        scat_wait((CHW - 1) % 4)
        gather_wait(CHW % 4)
        gather_wait((CHW + 1) % 4)
        idx_wait((CHW + 2) % 4)
        plsc.subcore_barrier()

        pltpu.async_copy(acc.at[pl.ds(s * RPT, RPT)],
                         out_hbm.at[c, pl.ds(s * RPT, RPT)], smg[0])

        @pl.when(s == NS - 1)
        def _():
            pltpu.async_copy(acc.at[pl.ds(NS * RPT, TAIL)],
                             out_hbm.at[c, pl.ds(NS * RPT, TAIL)], smg[1])

        prev_out.append(out_hbm)

    dump_wait(prev_out.pop())


def _seg3(xa, xb, si_ii, di_ii, si_c, di_c, si_cb, di_cb):
    mesh = plsc.VectorSubcoreMesh(core_axis_name="c", subcore_axis_name="s")
    f = functools.partial(
        pl.kernel,
        mesh=mesh,
        out_type=[jax.ShapeDtypeStruct((NC, ND, D), _f32)] * 3,
        scratch_types=[
            pltpu.VMEM_SHARED((ND, D), _f32),
            pltpu.VMEM((ZR, D), _f32),
            pltpu.VMEM((K,), jnp.int32),
            pltpu.VMEM((K,), jnp.int32),
            pltpu.VMEM((K,), jnp.int32),
            pltpu.VMEM((K,), jnp.int32),
            pltpu.VMEM((K,), jnp.int32),
            pltpu.VMEM((K,), jnp.int32),
            pltpu.VMEM((K,), jnp.int32),
            pltpu.VMEM((K,), jnp.int32),
            pltpu.VMEM((K, D), _f32),
            pltpu.VMEM((K, D), _f32),
            pltpu.VMEM((K, D), _f32),
            pltpu.VMEM((K, D), _f32),
        ] + [pltpu.SemaphoreType.DMA] * 16,
    )(_seg3_body)
    return f(xa, xb, si_ii, di_ii, si_c, di_c, si_cb, di_cb)


_RB = 1000


def _dense_body(aii, acb, ac, xd, xs, wri, wrcb, wrc, wrd, wrs, bd, bs,
                d_o, s_o):
    agg_ii = aii[0] + aii[1]
    agg_cb = acb[0] + acb[1]
    agg_c = ac[0] + ac[1]
    d = (jnp.dot(agg_ii, wri[...], preferred_element_type=_f32)
         + jnp.dot(agg_cb, wrcb[...], preferred_element_type=_f32)
         + jnp.dot(xd[...], wrd[...], preferred_element_type=_f32)
         + bd[...])
    s = (jnp.dot(agg_c, wrc[...], preferred_element_type=_f32)
         + jnp.dot(xs[...], wrs[...], preferred_element_type=_f32)
         + bs[...])
    d_o[...] = jnp.where(d >= 0, d, 0.01 * d)
    s_o[...] = jnp.where(s >= 0, s, 0.01 * s)


def _dense(p_ii, p_cb, p_c, xd, xs, wri, wrcb, wrc, wrd, wrs, bd, bs):
    n = ND // _RB
    part = pl.BlockSpec((2, _RB, D), lambda i: (0, i, 0))
    row = pl.BlockSpec((_RB, D), lambda i: (i, 0))
    mat = pl.BlockSpec((D, D), lambda i: (0, 0))
    vec = pl.BlockSpec((1, D), lambda i: (0, 0))
    return pl.pallas_call(
        _dense_body,
        grid=(n,),
        in_specs=[part, part, part, row, row, mat, mat, mat, mat, mat,
                  vec, vec],
        out_specs=[row, row],
        out_shape=[jax.ShapeDtypeStruct((ND, D), _f32)] * 2,
    )(p_ii, p_cb, p_c, xd, xs, wri, wrcb, wrc, wrd, wrs, bd, bs)


_NB = ND // _RB


def _dense2bn_body(aii, acb, ac, xd, xs, wri, wrcb, wrc, wrd, wrs, bd, bs,
                   g, bb, d_o, s_o, draw, sraw, stat):
    i = pl.program_id(0)

    @pl.when(i < _NB)
    def _():
        agg_ii = aii[0] + aii[1]
        agg_cb = acb[0] + acb[1]
        agg_c = ac[0] + ac[1]
        d = (jnp.dot(agg_ii, wri[...], preferred_element_type=_f32)
             + jnp.dot(agg_cb, wrcb[...], preferred_element_type=_f32)
             + jnp.dot(xd[...], wrd[...], preferred_element_type=_f32)
             + bd[...])
        s = (jnp.dot(agg_c, wrc[...], preferred_element_type=_f32)
             + jnp.dot(xs[...], wrs[...], preferred_element_type=_f32)
             + bs[...])
        d = jnp.where(d >= 0, d, 0.01 * d)
        s = jnp.where(s >= 0, s, 0.01 * s)

        @pl.when(i == 0)
        def _():
            stat[...] = jnp.zeros((8, D), _f32)

        draw[pl.ds(i * _RB, _RB), :] = d
        sraw[pl.ds(i * _RB, _RB), :] = s
        stat[0:1, :] += jnp.sum(d, axis=0, keepdims=True)
        stat[1:2, :] += jnp.sum(d * d, axis=0, keepdims=True)
        stat[2:3, :] += jnp.sum(s, axis=0, keepdims=True)
        stat[3:4, :] += jnp.sum(s * s, axis=0, keepdims=True)

    @pl.when(i >= _NB)
    def _():
        k = i - _NB
        gv = g[...]
        bv = bb[...]
        for raw, r0, o in ((draw, 0, d_o), (sraw, 2, s_o)):
            xv = raw[pl.ds(k * _RB, _RB), :]
            m = stat[r0:r0 + 1, :] * (1.0 / ND)
            v = stat[r0 + 1:r0 + 2, :] * (1.0 / ND) - m * m
            o[...] = (xv - m) * lax.rsqrt(v + 1e-5) * gv + bv


def _dense2bn(p_ii, p_cb, p_c, xd, xs, wri, wrcb, wrc, wrd, wrs, bd, bs,
              gamma, beta):
    part = pl.BlockSpec((2, _RB, D), lambda i: (0, jnp.minimum(i, _NB - 1), 0))
    row = pl.BlockSpec((_RB, D), lambda i: (jnp.minimum(i, _NB - 1), 0))
    mat = pl.BlockSpec((D, D), lambda i: (0, 0))
    vec = pl.BlockSpec((1, D), lambda i: (0, 0))
    orow = pl.BlockSpec((_RB, D), lambda i: (jnp.maximum(i - _NB, 0), 0))
    return pl.pallas_call(
        _dense2bn_body,
        grid=(2 * _NB,),
        in_specs=[part, part, part, row, row, mat, mat, mat, mat, mat,
                  vec, vec, vec, vec],
        out_specs=[orow, orow],
        out_shape=[jax.ShapeDtypeStruct((ND, D), _f32)] * 2,
        scratch_shapes=[
            pltpu.VMEM((ND, D), _f32),
            pltpu.VMEM((ND, D), _f32),
            pltpu.VMEM((8, D), _f32),
        ],
    )(p_ii, p_cb, p_c, xd, xs, wri, wrcb, wrc, wrd, wrs, bd, bs,
      gamma, beta)


def kernel(x_drug, x_se, ei_interacts, ei_causes, ei_caused_by,
           W1_rel_ii, b1_rel_ii, W1_root_ii, W1_rel_c, b1_rel_c, W1_root_c,
           W1_rel_cb, b1_rel_cb, W1_root_cb,
           W2_rel_ii, b2_rel_ii, W2_root_ii, W2_rel_c, b2_rel_c, W2_root_c,
           W2_rel_cb, b2_rel_cb, W2_root_cb, bn_gamma, bn_beta):
    si_ii, di_ii = ei_interacts[0], ei_interacts[1]
    si_c, di_c = ei_causes[0], ei_causes[1]
    si_cb, di_cb = ei_caused_by[0], ei_caused_by[1]

    w1rd = W1_root_ii + W1_root_cb
    b1d = (b1_rel_ii + b1_rel_cb).reshape(1, D)
    b1s = b1_rel_c.reshape(1, D)
    w2rd = W2_root_ii + W2_root_cb
    b2d = (b2_rel_ii + b2_rel_cb).reshape(1, D)
    b2s = b2_rel_c.reshape(1, D)

    p_ii, p_c, p_cb = _seg3(x_drug, x_se, si_ii, di_ii, si_c, di_c,
                            si_cb, di_cb)
    d1, s1 = _dense(p_ii, p_cb, p_c, x_drug, x_se,
                    W1_rel_ii, W1_rel_cb, W1_rel_c, w1rd, W1_root_c,
                    b1d, b1s)
    q_ii, q_c, q_cb = _seg3(d1, s1, si_ii, di_ii, si_c, di_c, si_cb, di_cb)
    return _dense2bn(q_ii, q_cb, q_c, d1, s1,
                     W2_rel_ii, W2_rel_cb, W2_rel_c, w2rd, W2_root_c,
                     b2d, b2s,
                     bn_gamma.reshape(1, D), bn_beta.reshape(1, D))


# speedup vs baseline: 1.3496x; 1.0140x over previous
import functools

import jax
import jax.numpy as jnp
from jax import lax
from jax.experimental import pallas as pl
from jax.experimental.pallas import tpu as pltpu
from jax.experimental.pallas import tpu_sc as plsc

ND = 10000
D = 128
E = 320000
NC = 2
NS = 16
NW = NC * NS
K = 64
CH_TOT = E // K
CHW = CH_TOT // NW
XTRA = CH_TOT - CHW * NW
RPT = 624
TAIL = ND - NS * RPT
ZR = 48

_f32 = jnp.float32


def _seg3_body(xa, xb, si_ii, di_ii, si_c, di_c, si_cb, di_cb,
               out_ii, out_c, out_cb,
               acc, zbuf, sidx0, didx0, sidx1, didx1, sidx2, didx2,
               sidx3, didx3, sidx4, didx4,
               rows0, rows1, rows2, rows3, rows4,
               smi0, smd0, smi1, smd1, smi2, smd2, smi3, smd3, smi4, smd4,
               smg0, smg1, smg2, smg3, smg4,
               sms0, sms1, sms2, sms3, sms4):
    c = lax.axis_index("c")
    s = lax.axis_index("s")
    wid = c * NS + s
    sbuf = (sidx0, sidx1, sidx2, sidx3, sidx4)
    dbuf = (didx0, didx1, didx2, didx3, didx4)
    rbuf = (rows0, rows1, rows2, rows3, rows4)
    smi = (smi0, smi1, smi2, smi3, smi4)
    smd = (smd0, smd1, smd2, smd3, smd4)
    smg = (smg0, smg1, smg2, smg3, smg4)
    sms = (sms0, sms1, sms2, sms3, sms4)

    z16 = jnp.zeros((16,), _f32)

    def zrow(r, carry):
        for j in range(D // 16):
            zbuf[r, pl.ds(j * 16, 16)] = z16
        return carry

    lax.fori_loop(0, ZR, zrow, 0)

    prev_out = []
    for x_hbm, si_hbm, di_hbm, out_hbm in (
        (xa, si_ii, di_ii, out_ii),
        (xa, si_c, di_c, out_c),
        (xb, si_cb, di_cb, out_cb),
    ):
        cbase = wid * CHW

        def idx_start(ch, b):
            off = pl.multiple_of(jnp.minimum(ch, CH_TOT - 1) * K, 8)
            pltpu.async_copy(si_hbm.at[pl.ds(off, K)], sbuf[b], smi[b])
            pltpu.async_copy(di_hbm.at[pl.ds(off, K)], dbuf[b], smd[b])

        def idx_wait(b):
            pltpu.make_async_copy(si_hbm.at[pl.ds(0, K)], sbuf[b],
                                  smi[b]).wait()
            pltpu.make_async_copy(di_hbm.at[pl.ds(0, K)], dbuf[b],
                                  smd[b]).wait()

        def gather_start(b):
            pltpu.async_copy(x_hbm.at[sbuf[b]], rbuf[b], smg[b])

        def gather_wait(b):
            pltpu.make_async_copy(x_hbm.at[pl.ds(0, K)], rbuf[b],
                                  smg[b]).wait()

        def scat_start(b):
            pltpu.async_copy(rbuf[b], acc.at[dbuf[b]], sms[b], add=True)

        def scat_wait(b):
            pltpu.make_async_copy(rbuf[b], acc.at[pl.ds(0, K)],
                                  sms[b]).wait()

        def dump_wait(o):
            pltpu.make_async_copy(acc.at[pl.ds(s * RPT, RPT)],
                                  o.at[c, pl.ds(s * RPT, RPT)],
                                  smg[0]).wait()

            @pl.when(s == NS - 1)
            def _():
                pltpu.make_async_copy(acc.at[pl.ds(NS * RPT, TAIL)],
                                      o.at[c, pl.ds(NS * RPT, TAIL)],
                                      smg[1]).wait()

        idx_start(cbase, 0)
        idx_start(cbase + 1, 1)
        idx_start(cbase + 2, 2)
        if prev_out:
            dump_wait(prev_out.pop())
        for j in range(RPT // ZR):
            pltpu.sync_copy(zbuf, acc.at[pl.ds(s * RPT + j * ZR, ZR)])

        @pl.when(s == NS - 1)
        def _():
            pltpu.sync_copy(zbuf.at[pl.ds(0, TAIL)],
                            acc.at[pl.ds(NS * RPT, TAIL)])

        idx_wait(0)
        gather_start(0)
        idx_wait(1)
        gather_start(1)
        plsc.subcore_barrier()

        @pl.when(s < XTRA // NC)
        def _():
            idx_start(NW * CHW + c * (XTRA // NC) + s, 4)
            idx_wait(4)
            gather_start(4)
            gather_wait(4)
            scat_start(4)
            scat_wait(4)

        gather_wait(0)
        scat_start(0)
        idx_start(cbase + 3, 3)
        idx_wait(2)
        gather_start(2)
        idx_start(cbase + 4, 4)
        idx_wait(3)
        gather_start(3)

        def quint(j, carry):
            i = 5 * j + 1
            for bp in range(5):
                ch = i + bp
                b = (1 + bp) % 5
                gather_wait(b)
                scat_wait((b + 4) % 5)
                scat_start(b)
                idx_start(cbase + ch + 4, (b + 4) % 5)
                idx_wait((b + 3) % 5)
                gather_start((b + 3) % 5)
            return carry

        lax.fori_loop(0, (CHW - 1) // 5, quint, 0)
        scat_wait((CHW - 1) % 5)
        gather_wait(CHW % 5)
        gather_wait((CHW + 1) % 5)
        gather_wait((CHW + 2) % 5)
        idx_wait((CHW + 3) % 5)
        plsc.subcore_barrier()

        pltpu.async_copy(acc.at[pl.ds(s * RPT, RPT)],
                         out_hbm.at[c, pl.ds(s * RPT, RPT)], smg[0])

        @pl.when(s == NS - 1)
        def _():
            pltpu.async_copy(acc.at[pl.ds(NS * RPT, TAIL)],
                             out_hbm.at[c, pl.ds(NS * RPT, TAIL)], smg[1])

        prev_out.append(out_hbm)

    dump_wait(prev_out.pop())


def _seg3(xa, xb, si_ii, di_ii, si_c, di_c, si_cb, di_cb):
    mesh = plsc.VectorSubcoreMesh(core_axis_name="c", subcore_axis_name="s")
    f = functools.partial(
        pl.kernel,
        mesh=mesh,
        out_type=[jax.ShapeDtypeStruct((NC, ND, D), _f32)] * 3,
        scratch_types=[
            pltpu.VMEM_SHARED((ND, D), _f32),
            pltpu.VMEM((ZR, D), _f32),
            pltpu.VMEM((K,), jnp.int32),
            pltpu.VMEM((K,), jnp.int32),
            pltpu.VMEM((K,), jnp.int32),
            pltpu.VMEM((K,), jnp.int32),
            pltpu.VMEM((K,), jnp.int32),
            pltpu.VMEM((K,), jnp.int32),
            pltpu.VMEM((K,), jnp.int32),
            pltpu.VMEM((K,), jnp.int32),
            pltpu.VMEM((K,), jnp.int32),
            pltpu.VMEM((K,), jnp.int32),
            pltpu.VMEM((K, D), _f32),
            pltpu.VMEM((K, D), _f32),
            pltpu.VMEM((K, D), _f32),
            pltpu.VMEM((K, D), _f32),
            pltpu.VMEM((K, D), _f32),
        ] + [pltpu.SemaphoreType.DMA] * 20,
    )(_seg3_body)
    return f(xa, xb, si_ii, di_ii, si_c, di_c, si_cb, di_cb)


_RB = 1000


def _dense_body(aii, acb, ac, xd, xs, wri, wrcb, wrc, wrd, wrs, bd, bs,
                d_o, s_o):
    agg_ii = aii[0] + aii[1]
    agg_cb = acb[0] + acb[1]
    agg_c = ac[0] + ac[1]
    d = (jnp.dot(agg_ii, wri[...], preferred_element_type=_f32)
         + jnp.dot(agg_cb, wrcb[...], preferred_element_type=_f32)
         + jnp.dot(xd[...], wrd[...], preferred_element_type=_f32)
         + bd[...])
    s = (jnp.dot(agg_c, wrc[...], preferred_element_type=_f32)
         + jnp.dot(xs[...], wrs[...], preferred_element_type=_f32)
         + bs[...])
    d_o[...] = jnp.where(d >= 0, d, 0.01 * d)
    s_o[...] = jnp.where(s >= 0, s, 0.01 * s)


def _dense(p_ii, p_cb, p_c, xd, xs, wri, wrcb, wrc, wrd, wrs, bd, bs):
    n = ND // _RB
    part = pl.BlockSpec((2, _RB, D), lambda i: (0, i, 0))
    row = pl.BlockSpec((_RB, D), lambda i: (i, 0))
    mat = pl.BlockSpec((D, D), lambda i: (0, 0))
    vec = pl.BlockSpec((1, D), lambda i: (0, 0))
    return pl.pallas_call(
        _dense_body,
        grid=(n,),
        in_specs=[part, part, part, row, row, mat, mat, mat, mat, mat,
                  vec, vec],
        out_specs=[row, row],
        out_shape=[jax.ShapeDtypeStruct((ND, D), _f32)] * 2,
    )(p_ii, p_cb, p_c, xd, xs, wri, wrcb, wrc, wrd, wrs, bd, bs)


_NB = ND // _RB


def _dense2bn_body(aii, acb, ac, xd, xs, wri, wrcb, wrc, wrd, wrs, bd, bs,
                   g, bb, d_o, s_o, draw, sraw, stat):
    i = pl.program_id(0)

    @pl.when(i < _NB)
    def _():
        agg_ii = aii[0] + aii[1]
        agg_cb = acb[0] + acb[1]
        agg_c = ac[0] + ac[1]
        d = (jnp.dot(agg_ii, wri[...], preferred_element_type=_f32)
             + jnp.dot(agg_cb, wrcb[...], preferred_element_type=_f32)
             + jnp.dot(xd[...], wrd[...], preferred_element_type=_f32)
             + bd[...])
        s = (jnp.dot(agg_c, wrc[...], preferred_element_type=_f32)
             + jnp.dot(xs[...], wrs[...], preferred_element_type=_f32)
             + bs[...])
        d = jnp.where(d >= 0, d, 0.01 * d)
        s = jnp.where(s >= 0, s, 0.01 * s)

        @pl.when(i == 0)
        def _():
            stat[...] = jnp.zeros((8, D), _f32)

        draw[pl.ds(i * _RB, _RB), :] = d
        sraw[pl.ds(i * _RB, _RB), :] = s
        stat[0:1, :] += jnp.sum(d, axis=0, keepdims=True)
        stat[1:2, :] += jnp.sum(d * d, axis=0, keepdims=True)
        stat[2:3, :] += jnp.sum(s, axis=0, keepdims=True)
        stat[3:4, :] += jnp.sum(s * s, axis=0, keepdims=True)

    @pl.when(i >= _NB)
    def _():
        k = i - _NB
        gv = g[...]
        bv = bb[...]
        for raw, r0, o in ((draw, 0, d_o), (sraw, 2, s_o)):
            xv = raw[pl.ds(k * _RB, _RB), :]
            m = stat[r0:r0 + 1, :] * (1.0 / ND)
            v = stat[r0 + 1:r0 + 2, :] * (1.0 / ND) - m * m
            o[...] = (xv - m) * lax.rsqrt(v + 1e-5) * gv + bv


def _dense2bn(p_ii, p_cb, p_c, xd, xs, wri, wrcb, wrc, wrd, wrs, bd, bs,
              gamma, beta):
    part = pl.BlockSpec((2, _RB, D), lambda i: (0, jnp.minimum(i, _NB - 1), 0))
    row = pl.BlockSpec((_RB, D), lambda i: (jnp.minimum(i, _NB - 1), 0))
    mat = pl.BlockSpec((D, D), lambda i: (0, 0))
    vec = pl.BlockSpec((1, D), lambda i: (0, 0))
    orow = pl.BlockSpec((_RB, D), lambda i: (jnp.maximum(i - _NB, 0), 0))
    return pl.pallas_call(
        _dense2bn_body,
        grid=(2 * _NB,),
        in_specs=[part, part, part, row, row, mat, mat, mat, mat, mat,
                  vec, vec, vec, vec],
        out_specs=[orow, orow],
        out_shape=[jax.ShapeDtypeStruct((ND, D), _f32)] * 2,
        scratch_shapes=[
            pltpu.VMEM((ND, D), _f32),
            pltpu.VMEM((ND, D), _f32),
            pltpu.VMEM((8, D), _f32),
        ],
    )(p_ii, p_cb, p_c, xd, xs, wri, wrcb, wrc, wrd, wrs, bd, bs,
      gamma, beta)


def kernel(x_drug, x_se, ei_interacts, ei_causes, ei_caused_by,
           W1_rel_ii, b1_rel_ii, W1_root_ii, W1_rel_c, b1_rel_c, W1_root_c,
           W1_rel_cb, b1_rel_cb, W1_root_cb,
           W2_rel_ii, b2_rel_ii, W2_root_ii, W2_rel_c, b2_rel_c, W2_root_c,
           W2_rel_cb, b2_rel_cb, W2_root_cb, bn_gamma, bn_beta):
    si_ii, di_ii = ei_interacts[0], ei_interacts[1]
    si_c, di_c = ei_causes[0], ei_causes[1]
    si_cb, di_cb = ei_caused_by[0], ei_caused_by[1]

    w1rd = W1_root_ii + W1_root_cb
    b1d = (b1_rel_ii + b1_rel_cb).reshape(1, D)
    b1s = b1_rel_c.reshape(1, D)
    w2rd = W2_root_ii + W2_root_cb
    b2d = (b2_rel_ii + b2_rel_cb).reshape(1, D)
    b2s = b2_rel_c.reshape(1, D)

    p_ii, p_c, p_cb = _seg3(x_drug, x_se, si_ii, di_ii, si_c, di_c,
                            si_cb, di_cb)
    d1, s1 = _dense(p_ii, p_cb, p_c, x_drug, x_se,
                    W1_rel_ii, W1_rel_cb, W1_rel_c, w1rd, W1_root_c,
                    b1d, b1s)
    q_ii, q_c, q_cb = _seg3(d1, s1, si_ii, di_ii, si_c, di_c, si_cb, di_cb)
    return _dense2bn(q_ii, q_cb, q_c, d1, s1,
                     W2_rel_ii, W2_rel_cb, W2_rel_c, w2rd, W2_root_c,
                     b2d, b2s,
                     bn_gamma.reshape(1, D), bn_beta.reshape(1, D))
